# baseline (speedup 1.0000x reference)
import functools

import jax
import jax.numpy as jnp
from jax import lax
from jax.experimental import pallas as pl
from jax.experimental.pallas import tpu as pltpu
from jax.experimental.pallas import tpu_sc as plsc

BATCH = 4096
N_GROUPS = 26
INNER = 64
OUT = 128
N_TOK = 100000
K_DIM = N_GROUPS * INNER

NC = 2
NS = 16
NW = NC * NS
CH_PER_W = INNER // NW


def _gather_channel_major(xt, tablet):
    mesh = plsc.VectorSubcoreMesh(core_axis_name="c", subcore_axis_name="s")

    @functools.partial(
        pl.kernel,
        out_type=jax.ShapeDtypeStruct((K_DIM * BATCH,), jnp.float32),
        mesh=mesh,
        scratch_types=[
            pltpu.VMEM((N_TOK,), jnp.float32),
            pltpu.VMEM((BATCH,), jnp.int32),
            pltpu.VMEM((BATCH,), jnp.int32),
            pltpu.VMEM((BATCH,), jnp.float32),
            pltpu.VMEM((BATCH,), jnp.float32),
            pltpu.SemaphoreType.DMA,
            pltpu.SemaphoreType.DMA,
            pltpu.SemaphoreType.DMA,
            pltpu.SemaphoreType.DMA,
        ],
        compiler_params=pltpu.CompilerParams(needs_layout_passes=False),
    )
    def gather_kernel(xf_hbm, tablet_hbm, at_hbm, chan_v, iv0, iv1, ov0, ov1,
                      is0, is1, os0, os1):
        wid = lax.axis_index("s") * NC + lax.axis_index("c")
        ivs, ovs = (iv0, iv1), (ov0, ov1)
        iss, oss = (is0, is1), (os0, os1)

        n_t = CH_PER_W * N_GROUPS
        pltpu.async_copy(xf_hbm.at[pl.ds(0, BATCH)], iv0, is0)

        def ubody(u, carry):
            for v in (0, 1):
                t = 2 * u + v
                g = lax.rem(t, N_GROUPS)
                ci = t // N_GROUPS
                c = wid * CH_PER_W + ci
                if v == 0:
                    @pl.when(lax.rem(u, N_GROUPS // 2) == 0)
                    def _():
                        pltpu.sync_copy(tablet_hbm.at[c], chan_v)

                pltpu.make_async_copy(xf_hbm.at[pl.ds(0, BATCH)], ivs[v], iss[v]).wait()
                if v == 0:
                    pltpu.async_copy(
                        xf_hbm.at[pl.ds(lax.rem(t + 1, N_GROUPS) * BATCH,
                                        BATCH)], ivs[1], iss[1])
                else:
                    @pl.when(t + 1 < n_t)
                    def _():
                        pltpu.async_copy(
                            xf_hbm.at[pl.ds(lax.rem(t + 1, N_GROUPS) * BATCH,
                                            BATCH)], ivs[0], iss[0])

                @pl.when(t >= 2)
                def _():
                    pltpu.make_async_copy(
                        ovs[v], at_hbm.at[pl.ds(0, BATCH)], oss[v]).wait()

                idx_v, out_v = ivs[v], ovs[v]

                @plsc.parallel_loop(0, BATCH, step=16, unroll=16)
                def chunk(i):
                    idx16 = idx_v[pl.ds(i, 16)]
                    out_v[pl.ds(i, 16)] = plsc.load_gather(chan_v, [idx16])

                pltpu.async_copy(
                    out_v, at_hbm.at[pl.ds((g * INNER + c) * BATCH, BATCH)],
                    oss[v])
            return carry

        lax.fori_loop(0, n_t // 2, ubody, 0)
        for v in (0, 1):
            pltpu.make_async_copy(
                        ovs[v], at_hbm.at[pl.ds(0, BATCH)], oss[v]).wait()

    return gather_kernel(xt.reshape(N_GROUPS * BATCH), tablet)


def _project_channel_major(at3, w):
    sb = 8

    def mm(a_ref, w_ref, o_ref):
        for s in range(sb):
            o_ref[pl.ds(s * 128, 128), :] = lax.dot_general(
                a_ref[:, s, :], w_ref[...], (((0,), (1,)), ((), ())),
                preferred_element_type=jnp.float32)

    return pl.pallas_call(
        mm,
        grid=(BATCH // (sb * 128),),
        in_specs=[
            pl.BlockSpec((K_DIM, sb, 128), lambda j: (0, j, 0)),
            pl.BlockSpec((OUT, K_DIM), lambda j: (0, 0)),
        ],
        out_specs=pl.BlockSpec((sb * 128, OUT), lambda j: (j, 0)),
        out_shape=jax.ShapeDtypeStruct((BATCH, OUT), jnp.float32),
    )(at3, w)


def kernel(x, table, W):
    xt = jnp.transpose(x.astype(jnp.int32))
    tablet = jnp.transpose(table)
    at = _gather_channel_major(xt, tablet)
    at3 = at.reshape(K_DIM, BATCH // 128, 128)
    return _project_channel_major(at3, W)


# speedup vs baseline: 1.7546x; 1.7546x over previous
import functools

import jax
import jax.numpy as jnp
from jax import lax
from jax.experimental import pallas as pl
from jax.experimental.pallas import tpu as pltpu
from jax.experimental.pallas import tpu_sc as plsc

BATCH = 4096
N_GROUPS = 26
INNER = 64
OUT = 128
N_TOK = 100000
K_DIM = N_GROUPS * INNER

NC = 2
NS = 16
NW = NC * NS
CH_PER_W = INNER // NW


def _gather_channel_major(xf, tablet):
    mesh = plsc.VectorSubcoreMesh(core_axis_name="c", subcore_axis_name="s")

    @functools.partial(
        pl.kernel,
        out_type=jax.ShapeDtypeStruct((K_DIM * BATCH,), jnp.float32),
        mesh=mesh,
        scratch_types=[
            pltpu.VMEM((N_TOK,), jnp.float32),
            pltpu.VMEM((BATCH,), jnp.int32),
            pltpu.VMEM((BATCH,), jnp.int32),
            pltpu.VMEM((BATCH,), jnp.float32),
            pltpu.VMEM((BATCH,), jnp.float32),
            pltpu.VMEM_SHARED((N_GROUPS * BATCH,), jnp.int32),
            pltpu.SemaphoreType.DMA,
            pltpu.SemaphoreType.DMA,
            pltpu.SemaphoreType.DMA,
            pltpu.SemaphoreType.DMA,
        ],
        compiler_params=pltpu.CompilerParams(needs_layout_passes=False),
    )
    def gather_kernel(xf_hbm, tablet_hbm, at_hbm, chan_v, iv0, iv1, ov0, ov1,
                      xf_s, is0, is1, os0, os1):
        wid = lax.axis_index("s") * NC + lax.axis_index("c")
        ivs, ovs = (iv0, iv1), (ov0, ov1)
        iss, oss = (is0, is1), (os0, os1)

        @pl.when(lax.axis_index("s") == 0)
        def _():
            pltpu.sync_copy(xf_hbm, xf_s)

        plsc.subcore_barrier()

        n_t = CH_PER_W * N_GROUPS
        pltpu.async_copy(xf_s.at[pl.ds(0, BATCH)], iv0, is0)

        def ubody(u, carry):
            for v in (0, 1):
                t = 2 * u + v
                g = lax.rem(t, N_GROUPS)
                ci = t // N_GROUPS
                c = wid * CH_PER_W + ci
                if v == 0:
                    @pl.when(lax.rem(u, N_GROUPS // 2) == 0)
                    def _():
                        pltpu.sync_copy(tablet_hbm.at[c], chan_v)

                pltpu.make_async_copy(
                    xf_s.at[pl.ds(0, BATCH)], ivs[v], iss[v]).wait()
                if v == 0:
                    pltpu.async_copy(
                        xf_s.at[pl.ds(lax.rem(t + 1, N_GROUPS) * BATCH,
                                      BATCH)], ivs[1], iss[1])
                else:
                    @pl.when(t + 1 < n_t)
                    def _():
                        pltpu.async_copy(
                            xf_s.at[pl.ds(lax.rem(t + 1, N_GROUPS) * BATCH,
                                          BATCH)], ivs[0], iss[0])

                @pl.when(t >= 2)
                def _():
                    pltpu.make_async_copy(
                        ovs[v], at_hbm.at[pl.ds(0, BATCH)], oss[v]).wait()

                idx_v, out_v = ivs[v], ovs[v]

                @plsc.parallel_loop(0, BATCH, step=16, unroll=16)
                def chunk(i):
                    idx16 = idx_v[pl.ds(i, 16)]
                    out_v[pl.ds(i, 16)] = plsc.load_gather(chan_v, [idx16])

                pltpu.async_copy(
                    out_v, at_hbm.at[pl.ds((g * INNER + c) * BATCH, BATCH)],
                    oss[v])
            return carry

        lax.fori_loop(0, n_t // 2, ubody, 0)
        for v in (0, 1):
            pltpu.make_async_copy(
                ovs[v], at_hbm.at[pl.ds(0, BATCH)], oss[v]).wait()

    return gather_kernel(xf, tablet)


def _project_channel_major(at3, w):
    sb = 8

    def mm(a_ref, w_ref, o_ref):
        a = jnp.reshape(a_ref[...], (K_DIM, sb * 128))
        o_ref[...] = lax.dot_general(
            a, w_ref[...], (((0,), (1,)), ((), ())),
            preferred_element_type=jnp.float32)

    return pl.pallas_call(
        mm,
        grid=(BATCH // (sb * 128),),
        in_specs=[
            pl.BlockSpec((K_DIM, sb, 128), lambda j: (0, j, 0)),
            pl.BlockSpec((OUT, K_DIM), lambda j: (0, 0)),
        ],
        out_specs=pl.BlockSpec((sb * 128, OUT), lambda j: (j, 0)),
        out_shape=jax.ShapeDtypeStruct((BATCH, OUT), jnp.float32),
    )(at3, w)


def kernel(x, table, W):
    xt = jnp.transpose(x.astype(jnp.int32))
    tablet = jnp.transpose(table)
    at = _gather_channel_major(xt.reshape(N_GROUPS * BATCH), tablet)
    at3 = at.reshape(K_DIM, BATCH // 128, 128)
    return _project_channel_major(at3, W)
